# baseline (speedup 1.0000x reference)
import functools

import jax
import jax.numpy as jnp
from jax import lax
from jax.experimental import pallas as pl
from jax.experimental.pallas import tpu as pltpu
from jax.experimental.pallas import tpu_sc as plsc

NUM_ACTIONS = 1000000
LATENT_DIM = 32
BATCH = 16384
HIST = 50

_NC = 2
_NS = 16
_NW = _NC * _NS

_NB = BATCH // _NW


def _gather_kernel(table_hbm, idxT_hbm, out_hbm,
                   idx0, idx1, rows0, rows1,
                   si0, si1, sg0, sg1, so0, so1):
    wid = lax.axis_index("s") * _NC + lax.axis_index("c")
    b0 = wid * _NB

    idx_v = (idx0, idx1)
    rows_v = (rows0, rows1)
    si = (si0, si1)
    sg = (sg0, sg1)
    so = (so0, so1)

    def idx_start(h, s):
        pltpu.async_copy(idxT_hbm.at[h, pl.ds(b0, _NB)], idx_v[s], si[s])

    def idx_wait(h, s):
        pltpu.make_async_copy(
            idxT_hbm.at[h, pl.ds(b0, _NB)], idx_v[s], si[s]).wait()

    def gather_start(s):
        pltpu.async_copy(table_hbm.at[idx_v[s]], rows_v[s], sg[s])

    def gather_wait(s):
        pltpu.make_async_copy(
            table_hbm.at[idx_v[s]], rows_v[s], sg[s]).wait()

    def out_start(h, s):
        pltpu.async_copy(rows_v[s], out_hbm.at[h, pl.ds(b0, _NB)], so[s])

    def out_wait(h, s):
        pltpu.make_async_copy(
            rows_v[s], out_hbm.at[h, pl.ds(b0, _NB)], so[s]).wait()

    def step(h, s, wait_so, next_gather, next_idx):
        gather_wait(s)
        if wait_so:
            out_wait(h - 2, s)
        if next_gather:
            idx_wait(h + 1, 1 - s)
            gather_start(1 - s)
        out_start(h, s)
        if next_idx:
            idx_start(h + 2, s)

    idx_start(0, 0)
    idx_wait(0, 0)
    gather_start(0)
    idx_start(1, 1)
    step(0, 0, False, True, True)
    step(1, 1, False, True, True)

    def body(j, carry):
        h = 2 * j
        step(h, 0, True, True, True)
        step(h + 1, 1, True, True, True)
        return carry

    lax.fori_loop(1, HIST // 2 - 1, body, 0)

    step(HIST - 2, 0, True, True, False)
    step(HIST - 1, 1, True, False, False)
    out_wait(HIST - 2, 0)
    out_wait(HIST - 1, 1)


@jax.jit
def _embed_lookup(idxT, table):
    mesh = plsc.VectorSubcoreMesh(core_axis_name="c", subcore_axis_name="s")
    kfn = functools.partial(
        pl.kernel,
        mesh=mesh,
        out_type=jax.ShapeDtypeStruct((HIST, BATCH, LATENT_DIM), jnp.float32),
        scratch_types=[
            pltpu.VMEM((_NB,), jnp.int32),
            pltpu.VMEM((_NB,), jnp.int32),
            pltpu.VMEM((_NB, LATENT_DIM), jnp.float32),
            pltpu.VMEM((_NB, LATENT_DIM), jnp.float32),
            pltpu.SemaphoreType.DMA,
            pltpu.SemaphoreType.DMA,
            pltpu.SemaphoreType.DMA,
            pltpu.SemaphoreType.DMA,
            pltpu.SemaphoreType.DMA,
            pltpu.SemaphoreType.DMA,
        ],
        compiler_params=pltpu.CompilerParams(
            use_tc_tiling_on_sc=False,
            disable_bounds_checks=True,
            disable_semaphore_checks=True,
            skip_device_barrier=True,
        ),
    )(_gather_kernel)
    return kfn(table, idxT)


def kernel(input, action_embedding):
    idxT = input.astype(jnp.int32).T
    out3 = _embed_lookup(idxT, action_embedding)
    return out3.transpose(1, 0, 2)


# speedup vs baseline: 1.0027x; 1.0027x over previous
import functools

import jax
import jax.numpy as jnp
from jax import lax
from jax.experimental import pallas as pl
from jax.experimental.pallas import tpu as pltpu
from jax.experimental.pallas import tpu_sc as plsc

NUM_ACTIONS = 1000000
LATENT_DIM = 32
BATCH = 16384
HIST = 50

_NC = 2
_NS = 16
_NW = _NC * _NS

_NB = BATCH // _NW


def _gather_kernel(table_hbm, idxT_hbm, out_hbm,
                   idx0, idx1, rows0, rows1,
                   si0, si1, sg0, sg1, so0, so1):
    wid = lax.axis_index("s") * _NC + lax.axis_index("c")
    b0 = wid * _NB

    idx_v = (idx0, idx1)
    rows_v = (rows0, rows1)
    si = (si0, si1)
    sg = (sg0, sg1)
    so = (so0, so1)

    def idx_start(h, s):
        pltpu.async_copy(idxT_hbm.at[h, pl.ds(b0, _NB)], idx_v[s], si[s])

    def idx_wait(h, s):
        pltpu.make_async_copy(
            idxT_hbm.at[h, pl.ds(b0, _NB)], idx_v[s], si[s]).wait()

    def gather_start(s):
        pltpu.async_copy(table_hbm.at[idx_v[s]], rows_v[s], sg[s])

    def gather_wait(s):
        pltpu.make_async_copy(
            table_hbm.at[idx_v[s]], rows_v[s], sg[s]).wait()

    def out_start(h, s):
        pltpu.async_copy(rows_v[s], out_hbm.at[h, pl.ds(b0, _NB)], so[s])

    def out_wait(h, s):
        pltpu.make_async_copy(
            rows_v[s], out_hbm.at[h, pl.ds(b0, _NB)], so[s]).wait()

    def step(h, s, wait_so, next_gather, next_idx):
        gather_wait(s)
        if wait_so:
            out_wait(h - 2, s)
        if next_gather:
            idx_wait(h + 1, 1 - s)
            gather_start(1 - s)
        out_start(h, s)
        if next_idx:
            idx_start(h + 2, s)

    idx_start(0, 0)
    idx_wait(0, 0)
    gather_start(0)
    idx_start(1, 1)
    step(0, 0, False, True, True)
    step(1, 1, False, True, True)

    def body(j, carry):
        h = 2 * j
        step(h, 0, True, True, True)
        step(h + 1, 1, True, True, True)
        return carry

    lax.fori_loop(1, HIST // 2 - 1, body, 0)

    step(HIST - 2, 0, True, True, False)
    step(HIST - 1, 1, True, False, False)
    out_wait(HIST - 2, 0)
    out_wait(HIST - 1, 1)


@jax.jit
def _embed_lookup(idxT, table):
    mesh = plsc.VectorSubcoreMesh(core_axis_name="c", subcore_axis_name="s")
    kfn = functools.partial(
        pl.kernel,
        mesh=mesh,
        out_type=jax.ShapeDtypeStruct((HIST, BATCH, LATENT_DIM), jnp.float32),
        scratch_types=[
            pltpu.VMEM((_NB,), jnp.int32),
            pltpu.VMEM((_NB,), jnp.int32),
            pltpu.VMEM((_NB, LATENT_DIM), jnp.float32),
            pltpu.VMEM((_NB, LATENT_DIM), jnp.float32),
            pltpu.SemaphoreType.DMA,
            pltpu.SemaphoreType.DMA,
            pltpu.SemaphoreType.DMA,
            pltpu.SemaphoreType.DMA,
            pltpu.SemaphoreType.DMA,
            pltpu.SemaphoreType.DMA,
        ],
        compiler_params=pltpu.CompilerParams(use_tc_tiling_on_sc=False),
    )(_gather_kernel)
    return kfn(table, idxT)


def kernel(input, action_embedding):
    idxT = input.astype(jnp.int32).T
    out3 = _embed_lookup(idxT, action_embedding)
    return out3.transpose(1, 0, 2)


# speedup vs baseline: 1.0028x; 1.0001x over previous
import functools

import jax
import jax.numpy as jnp
from jax import lax
from jax.experimental import pallas as pl
from jax.experimental.pallas import tpu as pltpu
from jax.experimental.pallas import tpu_sc as plsc

NUM_ACTIONS = 1000000
LATENT_DIM = 32
BATCH = 16384
HIST = 50

_NC = 2
_NS = 16
_NW = _NC * _NS

_NB = BATCH // _NW


def _gather_kernel(table_hbm, idxT_hbm, out_hbm,
                   idx0, idx1, rows0, rows1,
                   si0, si1, sg0, sg1, so0, so1):
    wid = lax.axis_index("s") * _NC + lax.axis_index("c")
    b0 = wid * _NB

    idx_v = (idx0, idx1)
    rows_v = (rows0, rows1)
    si = (si0, si1)
    sg = (sg0, sg1)
    so = (so0, so1)

    def idx_start(h, s):
        pltpu.async_copy(idxT_hbm.at[h, pl.ds(b0, _NB)], idx_v[s], si[s])

    def idx_wait(h, s):
        pltpu.make_async_copy(
            idxT_hbm.at[h, pl.ds(b0, _NB)], idx_v[s], si[s]).wait()

    def gather_start(s):
        pltpu.async_copy(table_hbm.at[idx_v[s]], rows_v[s], sg[s])

    def gather_wait(s):
        pltpu.make_async_copy(
            table_hbm.at[idx_v[s]], rows_v[s], sg[s]).wait()

    def out_start(h, s):
        pltpu.async_copy(rows_v[s], out_hbm.at[h, pl.ds(b0, _NB)], so[s])

    def out_wait(h, s):
        pltpu.make_async_copy(
            rows_v[s], out_hbm.at[h, pl.ds(b0, _NB)], so[s]).wait()

    def step(h, s, wait_prev_store, next_gather, next_idx):
        gather_wait(s)
        if wait_prev_store:
            out_wait(h - 1, 1 - s)
        if next_gather:
            idx_wait(h + 1, 1 - s)
            gather_start(1 - s)
        out_start(h, s)
        if next_idx:
            idx_start(h + 2, s)

    idx_start(0, 0)
    idx_wait(0, 0)
    gather_start(0)
    idx_start(1, 1)
    step(0, 0, False, True, True)

    def body(j, carry):
        h = 2 * j + 1
        step(h, 1, True, True, True)
        step(h + 1, 0, True, True, True)
        return carry

    step(1, 1, True, True, True)
    step(2, 0, True, True, True)
    lax.fori_loop(1, HIST // 2 - 2, body, 0)

    step(HIST - 3, 1, True, True, True)
    step(HIST - 2, 0, True, True, False)
    step(HIST - 1, 1, True, False, False)
    out_wait(HIST - 1, 1)


@jax.jit
def _embed_lookup(idxT, table):
    mesh = plsc.VectorSubcoreMesh(core_axis_name="c", subcore_axis_name="s")
    kfn = functools.partial(
        pl.kernel,
        mesh=mesh,
        out_type=jax.ShapeDtypeStruct((HIST, BATCH, LATENT_DIM), jnp.float32),
        scratch_types=[
            pltpu.VMEM((_NB,), jnp.int32),
            pltpu.VMEM((_NB,), jnp.int32),
            pltpu.VMEM((_NB, LATENT_DIM), jnp.float32),
            pltpu.VMEM((_NB, LATENT_DIM), jnp.float32),
            pltpu.SemaphoreType.DMA,
            pltpu.SemaphoreType.DMA,
            pltpu.SemaphoreType.DMA,
            pltpu.SemaphoreType.DMA,
            pltpu.SemaphoreType.DMA,
            pltpu.SemaphoreType.DMA,
        ],
        compiler_params=pltpu.CompilerParams(use_tc_tiling_on_sc=False),
    )(_gather_kernel)
    return kfn(table, idxT)


def kernel(input, action_embedding):
    idxT = input.astype(jnp.int32).T
    out3 = _embed_lookup(idxT, action_embedding)
    return out3.transpose(1, 0, 2)
